# baseline (speedup 1.0000x reference)
import functools

import jax
import jax.numpy as jnp
from jax import lax
from jax.experimental import pallas as pl
from jax.experimental.pallas import tpu as pltpu
from jax.experimental.pallas import tpu_sc as plsc

F32 = jnp.float32
BF16 = jnp.bfloat16
EPS = 1e-5


def _cp():
    return pltpu.CompilerParams(vmem_limit_bytes=100 * 1024 * 1024)


def _mm(a, b, precision=None):
    return lax.dot_general(a, b, (((1,), (0,)), ((), ())),
                           preferred_element_type=F32, precision=precision)


def _stats_rows(pairs, cout):
    rows = [v[None, :] for v in pairs]
    rows.append(jnp.zeros((8 - len(rows), cout), F32))
    return jnp.concatenate(rows, axis=0)


def _colsum(q):
    return jnp.sum(q, axis=1)


def _colsum_bf(q):
    one = jnp.ones((q.shape[1], 8), BF16)
    return lax.dot_general(q, one, (((1,), (0,)), ((), ())),
                           preferred_element_type=F32)[:, 0]


def _phase_stats_bf(qc, qg):
    cout = qc[0].shape[0]
    zs = jnp.zeros((cout,), F32)
    sums = [zs, zs, zs, zs]
    for q in qc:
        sums[0] = sums[0] + _colsum_bf(q)
        sums[1] = sums[1] + _colsum_bf(q * q)
    for q in qg:
        sums[2] = sums[2] + _colsum_bf(q)
        sums[3] = sums[3] + _colsum_bf(q * q)
    return _stats_rows(sums, cout)


def _accum_stats(st_ref, s):
    b = pl.program_id(0)

    @pl.when(b == 0)
    def _():
        st_ref[...] = s

    @pl.when(b != 0)
    def _():
        st_ref[...] = st_ref[...] + s


def _bn_coeffs(st_ref, g_ref, b_ref, row, inv):
    m = st_ref[row] * inv
    v = st_ref[row + 1] * inv - m * m
    s = g_ref[0] * lax.rsqrt(v + EPS)
    t = b_ref[0] - m * s
    return s, t


def _gate(c, g, sc, tc, sg, tg):
    cn = c * sc[:, None] + tc[:, None]
    gn = g * sg[:, None] + tg[:, None]
    return jnp.tanh(cn) * jax.nn.sigmoid(gn)


def _conv_stats(c, g):
    return _stats_rows([_colsum(c), _colsum(c * c), _colsum(g), _colsum(g * g)], c.shape[0])


def _phase_conv(p_list, w_ref, n_out, cond=None):
    P = len(p_list)
    Q = P // 2
    cin = p_list[0].shape[0]
    zcol = jnp.zeros((cin, 1), F32)
    shifted = {}

    def shift(j):
        if j not in shifted:
            shifted[j] = jnp.concatenate([p_list[j][:, 1:], zcol], axis=1)
        return shifted[j]

    qs = []
    for r in range(Q):
        acc = None
        for k in range(4):
            j = (2 * r + k) % P
            src = shift(j) if (2 * r + k) >= P else p_list[j]
            term = _mm(w_ref[k], src[:, :n_out])
            acc = term if acc is None else acc + term
        if cond is not None:
            acc = acc + cond
        qs.append(acc)
    return qs


def _zero_tail(q, valid):
    if valid >= q.shape[1]:
        return q
    ii = lax.broadcasted_iota(jnp.int32, q.shape, 1)
    return jnp.where(ii < valid, q, 0.0)


def _phase_valids(tout, Q, n_out):
    return [min(n_out, -(-(tout - r) // Q)) for r in range(Q)]


def _enc1_body(tout, x_ref, wc_ref, wg_ref, *out_refs):
    P = x_ref.shape[1]
    p_list = [x_ref[0, j] for j in range(P)]
    Q = P // 2
    n_out = out_refs[0].shape[2]
    valids = _phase_valids(tout, Q, n_out)
    qc = _phase_conv(p_list, wc_ref, n_out)
    qg = _phase_conv(p_list, wg_ref, n_out)
    qc = [_zero_tail(q, v) for q, v in zip(qc, valids)]
    qg = [_zero_tail(q, v) for q, v in zip(qg, valids)]
    for i, q in enumerate(qc + qg):
        out_refs[i][0] = q
    out_refs[-1][0] = _phase_stats(qc, qg)


def _phase_stats(qc, qg):
    cout = qc[0].shape[0]
    zs = jnp.zeros((cout,), F32)
    sums = [zs, zs, zs, zs]
    for q in qc:
        sums[0] = sums[0] + _colsum(q)
        sums[1] = sums[1] + _colsum(q * q)
    for q in qg:
        sums[2] = sums[2] + _colsum(q)
        sums[3] = sums[3] + _colsum(q * q)
    return _stats_rows(sums, cout)


def _enc1(x8, wc, wg, tout, n_out):
    B, P, C, n = x8.shape
    cout = wc.shape[1]
    Q = P // 2
    wspec = pl.BlockSpec((4, cout, C), lambda b: (0, 0, 0))
    ospec = pl.BlockSpec((1, cout, n_out), lambda b: (b, 0, 0))
    oshape = jax.ShapeDtypeStruct((B, cout, n_out), F32)
    res = pl.pallas_call(
        functools.partial(_enc1_body, tout),
        grid=(B,),
        in_specs=[pl.BlockSpec((1, P, C, n), lambda b: (b, 0, 0, 0)),
                  wspec, wspec],
        out_specs=[ospec] * (2 * Q) + [pl.BlockSpec((1, 8, cout),
                                                    lambda b: (b, 0, 0))],
        out_shape=[oshape] * (2 * Q) + [jax.ShapeDtypeStruct((B, 8, cout),
                                                             F32)],
        compiler_params=pltpu.CompilerParams(
            vmem_limit_bytes=100 * 1024 * 1024,
            dimension_semantics=("parallel",)),
    )(x8, wc, wg)
    return res[:Q], res[Q:2 * Q], res[2 * Q]


def _enc_body(tout, count_prev, nph, n_out, *refs):
    cps = refs[:nph]
    gps = refs[nph:2 * nph]
    st_ref, gc_ref, bc_ref, gg_ref, bg_ref, wc_ref, wg_ref = \
        refs[2 * nph:2 * nph + 7]
    out_refs = refs[2 * nph + 7:]
    inv = 1.0 / count_prev
    st = st_ref[...]
    if st.ndim == 3:
        st = jnp.sum(st, axis=0)
    sc, tc = _bn_coeffs(st, gc_ref, bc_ref, 0, inv)
    sg, tg = _bn_coeffs(st, gg_ref, bg_ref, 2, inv)
    p_list = [_gate(cps[j][0], gps[j][0], sc, tc, sg, tg) for j in range(nph)]
    Q = nph // 2
    valids = _phase_valids(tout, Q, n_out)
    qc = _phase_conv(p_list, wc_ref, n_out)
    qg = _phase_conv(p_list, wg_ref, n_out)
    qc = [_zero_tail(q, v) for q, v in zip(qc, valids)]
    qg = [_zero_tail(q, v) for q, v in zip(qg, valids)]
    for i, q in enumerate(qc + qg):
        out_refs[i][0] = q
    _accum_stats(out_refs[-1], _phase_stats(qc, qg))


def _enc_fused(cps, gps, st, bn, wc, wg, tout, n_out, count_prev):
    nph = len(cps)
    B, C, n = cps[0].shape
    cout = wc.shape[1]
    Q = nph // 2
    gc, bc, gg, bg = bn
    wspec = pl.BlockSpec((4, cout, C), lambda b: (0, 0, 0))
    dspec = pl.BlockSpec((1, C, n), lambda b: (b, 0, 0))
    pspec = pl.BlockSpec((1, C), lambda b: (0, 0))
    ospec = pl.BlockSpec((1, cout, n_out), lambda b: (b, 0, 0))
    oshape = jax.ShapeDtypeStruct((B, cout, n_out), F32)
    sspec = (pl.BlockSpec((B, 8, C), lambda b: (0, 0, 0)) if st.ndim == 3
             else pl.BlockSpec((8, C), lambda b: (0, 0)))
    res = pl.pallas_call(
        functools.partial(_enc_body, tout, float(count_prev), nph, n_out),
        grid=(B,),
        in_specs=[dspec] * (2 * nph) + [
            sspec, pspec, pspec, pspec, pspec, wspec, wspec],
        out_specs=[ospec] * (2 * Q) + [pl.BlockSpec((8, cout),
                                                    lambda b: (0, 0))],
        out_shape=[oshape] * (2 * Q) + [jax.ShapeDtypeStruct((8, cout), F32)],
        compiler_params=_cp(),
    )(*cps, *gps, st, gc[None], bc[None], gg[None], bg[None], wc, wg)
    return res[:Q], res[Q:2 * Q], res[2 * Q]


def _latent_body(count_prev, cp_ref, gp_ref, st_ref, gc_ref, bc_ref,
                 gg_ref, bg_ref, w_ref, z_ref, sto_ref):
    inv = 1.0 / count_prev
    sc, tc = _bn_coeffs(st_ref, gc_ref, bc_ref, 0, inv)
    sg, tg = _bn_coeffs(st_ref, gg_ref, bg_ref, 2, inv)
    act = _gate(cp_ref[0], gp_ref[0], sc, tc, sg, tg)
    z = _mm(w_ref[...], act)
    z_ref[0] = z
    _accum_stats(sto_ref, _stats_rows(
        [_colsum(z), _colsum(z * z)], z.shape[0]))


def _latent(cp, gp, st, bn, w, count_prev):
    B, C, T = cp.shape
    cout = w.shape[0]
    gc, bc, gg, bg = bn
    dspec = pl.BlockSpec((1, C, T), lambda b: (b, 0, 0))
    pspec = pl.BlockSpec((1, C), lambda b: (0, 0))
    return pl.pallas_call(
        functools.partial(_latent_body, float(count_prev)),
        grid=(B,),
        in_specs=[dspec, dspec, pl.BlockSpec((8, C), lambda b: (0, 0)),
                  pspec, pspec, pspec, pspec,
                  pl.BlockSpec((cout, C), lambda b: (0, 0))],
        out_specs=[pl.BlockSpec((1, cout, T), lambda b: (b, 0, 0)),
                   pl.BlockSpec((8, cout), lambda b: (0, 0))],
        out_shape=[jax.ShapeDtypeStruct((B, cout, T), F32),
                   jax.ShapeDtypeStruct((8, cout), F32)],
        compiler_params=_cp(),
    )(cp, gp, st, gc[None], bc[None], gg[None], bg[None], w)


def _vq_body(count, z_ref, st_ref, lg_ref, lb_ref, cb_ref, idx_ref):
    inv = 1.0 / count
    m = st_ref[0] * inv
    v = st_ref[1] * inv - m * m
    sc = lg_ref[0] * lax.rsqrt(v + EPS)
    tc = lb_ref[0] - m * sc
    z = z_ref[0] * sc[:, None] + tc[:, None]
    cb = cb_ref[...]
    s = _mm(cb, z, precision=lax.Precision.HIGHEST)
    cn2 = jnp.sum(cb * cb, axis=1)
    val = cn2[:, None] - 2.0 * s
    mn = jnp.min(val, axis=0)
    ii = lax.broadcasted_iota(jnp.int32, val.shape, 0)
    idx = jnp.min(jnp.where(val <= mn[None, :], ii, jnp.int32(1 << 30)), axis=0)
    idx_ref[0, 0] = idx


def _vq_argmin(z, st, lg, lb, cb, count):
    B, D, T = z.shape
    n = cb.shape[0]
    return pl.pallas_call(
        functools.partial(_vq_body, float(count)),
        grid=(B,),
        in_specs=[pl.BlockSpec((1, D, T), lambda b: (b, 0, 0)),
                  pl.BlockSpec((8, D), lambda b: (0, 0)),
                  pl.BlockSpec((1, D), lambda b: (0, 0)),
                  pl.BlockSpec((1, D), lambda b: (0, 0)),
                  pl.BlockSpec((n, D), lambda b: (0, 0))],
        out_specs=pl.BlockSpec((1, 1, T), lambda b: (b, 0, 0)),
        out_shape=jax.ShapeDtypeStruct((B, 1, T), jnp.int32),
        compiler_params=_cp(),
    )(z, st, lg[None], lb[None], cb)


def _sc_gather(cb, idx, spk, emb):
    info = plsc.get_sparse_core_info()
    nc, ns = info.num_cores, info.num_subcores
    nw = nc * ns
    nidx = idx.shape[0]
    bpw = nidx // nw
    d = cb.shape[1]
    bsp = spk.shape[0]
    mesh = plsc.VectorSubcoreMesh(core_axis_name="c", subcore_axis_name="s")

    @functools.partial(
        pl.kernel,
        out_type=[jax.ShapeDtypeStruct((nidx, d), F32),
                  jax.ShapeDtypeStruct((bsp, emb.shape[1]), F32)],
        mesh=mesh,
        scratch_types=[pltpu.VMEM((bpw,), jnp.int32),
                       pltpu.VMEM((bpw, d), F32),
                       pltpu.VMEM((bsp,), jnp.int32),
                       pltpu.VMEM((bsp, emb.shape[1]), F32),
                       pltpu.SemaphoreType.DMA],
    )
    def k(cb_hbm, idx_hbm, spk_hbm, emb_hbm, zq_hbm, h_hbm,
          idx_v, rows_v, sidx_v, srows_v, sem):
        wid = lax.axis_index("s") * nc + lax.axis_index("c")
        base = wid * bpw
        pltpu.sync_copy(idx_hbm.at[pl.ds(base, bpw)], idx_v)
        pltpu.async_copy(cb_hbm.at[idx_v], rows_v, sem).wait()
        pltpu.sync_copy(rows_v, zq_hbm.at[pl.ds(base, bpw)])

        @pl.when(wid == 0)
        def _():
            pltpu.sync_copy(spk_hbm, sidx_v)
            pltpu.async_copy(emb_hbm.at[sidx_v], srows_v, sem).wait()
            pltpu.sync_copy(srows_v, h_hbm)

    return k(cb, idx, spk, emb)


def _gen1_body(mc_ref, mg_ref, x_ref, h_ref, cw_ref,
               ce_ref, co_ref, ge_ref, go_ref, st_ref):
    x = x_ref[0]
    cin, t = x.shape
    z1 = jnp.zeros((cin, 1), F32)
    xm = jnp.concatenate([z1, x[:, :t - 1]], axis=1)
    xp = jnp.concatenate([x[:, 1:], z1], axis=1)
    cond = _mm(cw_ref[...], h_ref[0, 0][:, None])
    ce = _mm(mc_ref[0], xm) + _mm(mc_ref[2], x) + cond
    co = _mm(mc_ref[1], x) + _mm(mc_ref[3], xp) + cond
    ge = _mm(mg_ref[0], xm) + _mm(mg_ref[2], x) + cond
    go = _mm(mg_ref[1], x) + _mm(mg_ref[3], xp) + cond
    ce_ref[0] = ce.astype(BF16)
    co_ref[0] = co.astype(BF16)
    ge_ref[0] = ge.astype(BF16)
    go_ref[0] = go.astype(BF16)
    s = _stats_rows(
        [_colsum(ce) + _colsum(co),
         _colsum(ce * ce) + _colsum(co * co),
         _colsum(ge) + _colsum(go),
         _colsum(ge * ge) + _colsum(go * go)], ce.shape[0])
    _accum_stats(st_ref, s)


def _gen1(x, h, mc, mg, cw):
    B, cin, T = x.shape
    cout = cw.shape[0]
    h3 = h[:, None, :]
    wspec = pl.BlockSpec((4, cout, cin), lambda b: (0, 0, 0))
    ospec = pl.BlockSpec((1, cout, T), lambda b: (b, 0, 0))
    oshape = jax.ShapeDtypeStruct((B, cout, T), BF16)
    return pl.pallas_call(
        _gen1_body,
        grid=(B,),
        in_specs=[wspec, wspec,
                  pl.BlockSpec((1, cin, T), lambda b: (b, 0, 0)),
                  pl.BlockSpec((1, 1, h.shape[1]), lambda b: (b, 0, 0)),
                  pl.BlockSpec((cout, h.shape[1]), lambda b: (0, 0))],
        out_specs=[ospec, ospec, ospec, ospec,
                   pl.BlockSpec((8, cout), lambda b: (0, 0))],
        out_shape=[oshape, oshape, oshape, oshape,
                   jax.ShapeDtypeStruct((8, cout), F32)],
        compiler_params=_cp(),
    )(mc, mg, x, h3, cw)


def _genf_body(nph, count_prev, *refs):
    cps = refs[:nph]
    gps = refs[nph:2 * nph]
    st_ref, gc_ref, bc_ref, gg_ref, bg_ref, mc_ref, mg_ref, h_ref, cw_ref = \
        refs[2 * nph:2 * nph + 9]
    outs = refs[2 * nph + 9:2 * nph + 9 + 4 * nph]
    sto_ref = refs[2 * nph + 9 + 4 * nph]

    inv = 1.0 / count_prev
    sc, tc = _bn_coeffs(st_ref, gc_ref, bc_ref, 0, inv)
    sg, tg = _bn_coeffs(st_ref, gg_ref, bg_ref, 2, inv)
    p = [_gate(cps[j][0], gps[j][0], sc, tc, sg, tg).astype(BF16)
         for j in range(nph)]
    cond = _mm(cw_ref[...], h_ref[0, 0][:, None])
    qc = [q.astype(BF16) for q in _tconv_phases(p, mc_ref, cond)]
    qg = [q.astype(BF16) for q in _tconv_phases(p, mg_ref, cond)]
    for i, q in enumerate(qc + qg):
        outs[i][0] = q
    _accum_stats(sto_ref, _phase_stats_bf(qc, qg))


def _act_body(nph, count_prev, *refs):
    cps = refs[:nph]
    gps = refs[nph:2 * nph]
    st_ref, gc_ref, bc_ref, gg_ref, bg_ref = refs[2 * nph:2 * nph + 5]
    outs = refs[2 * nph + 5:]
    inv = 1.0 / count_prev
    sc, tc = _bn_coeffs(st_ref, gc_ref, bc_ref, 0, inv)
    sg, tg = _bn_coeffs(st_ref, gg_ref, bg_ref, 2, inv)
    for j in range(nph):
        outs[j][0] = _gate(cps[j][0], gps[j][0], sc, tc, sg, tg).astype(BF16)


def _act_phases(cps, gps, st, bn, count_prev):
    nph = len(cps)
    B, C, T = cps[0].shape
    gc, bc, gg, bg = bn
    dspec = pl.BlockSpec((1, C, T), lambda b: (b, 0, 0))
    pspec = pl.BlockSpec((1, C), lambda b: (0, 0))
    oshape = jax.ShapeDtypeStruct((B, C, T), BF16)
    return pl.pallas_call(
        functools.partial(_act_body, nph, float(count_prev)),
        grid=(B,),
        in_specs=[dspec] * (2 * nph) + [
            pl.BlockSpec((8, C), lambda b: (0, 0)),
            pspec, pspec, pspec, pspec],
        out_specs=[dspec] * nph,
        out_shape=[oshape] * nph,
        compiler_params=_cp(),
    )(*cps, *gps, st, gc[None], bc[None], gg[None], bg[None])


def _tconv_phases(p_list, m_ref, cond):
    nph = len(p_list)
    cin = p_list[0].shape[0]
    z1 = jnp.zeros((cin, 1), p_list[0].dtype)
    pm = jnp.concatenate([z1, p_list[nph - 1][:, :-1]], axis=1)
    pp = jnp.concatenate([p_list[0][:, 1:], z1], axis=1)
    qs = []
    for j in range(nph):
        a = pm if j == 0 else p_list[j - 1]
        b = pp if j == nph - 1 else p_list[j + 1]
        qs.append(_mm(m_ref[0], a) + _mm(m_ref[2], p_list[j]) + cond)
        qs.append(_mm(m_ref[1], p_list[j]) + _mm(m_ref[3], b) + cond)
    return qs


def _gen_half_body(nph, *refs):
    ps = refs[:nph]
    m_ref, h_ref, cw_ref = refs[nph:nph + 3]
    outs = refs[nph + 3:nph + 3 + 2 * nph]
    sto_ref = refs[nph + 3 + 2 * nph]
    cond = _mm(cw_ref[...], h_ref[0, 0][:, None])
    qs = _tconv_phases([ps[j][0] for j in range(nph)], m_ref, cond)
    for i, q in enumerate(qs):
        outs[i][0] = q
    cout = qs[0].shape[0]
    zs = jnp.zeros((cout,), F32)
    s0, s1 = zs, zs
    for q in qs:
        s0 = s0 + _colsum(q)
        s1 = s1 + _colsum(q * q)
    _accum_stats(sto_ref, _stats_rows([s0, s1], cout))


def _gen_half(ps, m, h, cw):
    nph = len(ps)
    B, C, T = ps[0].shape
    cout = cw.shape[0]
    h3 = h[:, None, :]
    dspec = pl.BlockSpec((1, C, T), lambda b: (b, 0, 0))
    ospec = pl.BlockSpec((1, cout, T), lambda b: (b, 0, 0))
    oshape = jax.ShapeDtypeStruct((B, cout, T), F32)
    res = pl.pallas_call(
        functools.partial(_gen_half_body, nph),
        grid=(B,),
        in_specs=[dspec] * nph + [
            pl.BlockSpec((4, cout, C), lambda b: (0, 0, 0)),
            pl.BlockSpec((1, 1, h.shape[1]), lambda b: (b, 0, 0)),
            pl.BlockSpec((cout, h.shape[1]), lambda b: (0, 0))],
        out_specs=[ospec] * (2 * nph) + [pl.BlockSpec((8, cout),
                                                      lambda b: (0, 0))],
        out_shape=[oshape] * (2 * nph) + [jax.ShapeDtypeStruct((8, cout), F32)],
        compiler_params=_cp(),
    )(*ps, m, h3, cw)
    return res[:2 * nph], res[2 * nph]


def _gen_fused(cps, gps, st, bn, mc, mg, h, cw, count_prev):
    nph = len(cps)
    B, C, T = cps[0].shape
    cout = cw.shape[0]
    gc, bc, gg, bg = bn
    h3 = h[:, None, :]
    dspec = pl.BlockSpec((1, C, T), lambda b: (b, 0, 0))
    pspec = pl.BlockSpec((1, C), lambda b: (0, 0))
    wspec = pl.BlockSpec((4, cout, C), lambda b: (0, 0, 0))
    ospec = pl.BlockSpec((1, cout, T), lambda b: (b, 0, 0))
    oshape = jax.ShapeDtypeStruct((B, cout, T), BF16)
    res = pl.pallas_call(
        functools.partial(_genf_body, nph, float(count_prev)),
        grid=(B,),
        in_specs=[dspec] * (2 * nph) + [
            pl.BlockSpec((8, C), lambda b: (0, 0)),
            pspec, pspec, pspec, pspec, wspec, wspec,
            pl.BlockSpec((1, 1, h.shape[1]), lambda b: (b, 0, 0)),
            pl.BlockSpec((cout, h.shape[1]), lambda b: (0, 0))],
        out_specs=[ospec] * (4 * nph) + [pl.BlockSpec((8, cout),
                                                      lambda b: (0, 0))],
        out_shape=[oshape] * (4 * nph) + [jax.ShapeDtypeStruct((8, cout), F32)],
        compiler_params=_cp(),
    )(*cps, *gps, st, gc[None], bc[None], gg[None], bg[None], mc, mg, h3, cw)
    qs, sto = res[:4 * nph], res[4 * nph]
    return qs[:2 * nph], qs[2 * nph:], sto


def _pre_body(nph, count_prev, *refs):
    cps = refs[:nph]
    gps = refs[nph:2 * nph]
    st_ref, gc_ref, bc_ref, gg_ref, bg_ref, w_ref = \
        refs[2 * nph:2 * nph + 6]
    outs = refs[2 * nph + 6:2 * nph + 6 + nph]
    sto_ref = refs[2 * nph + 6 + nph]
    inv = 1.0 / count_prev
    sc, tc = _bn_coeffs(st_ref, gc_ref, bc_ref, 0, inv)
    sg, tg = _bn_coeffs(st_ref, gg_ref, bg_ref, 2, inv)
    cout = w_ref.shape[0]
    zs = jnp.zeros((cout,), F32)
    s0, s1 = zs, zs
    for j in range(nph):
        a = _gate(cps[j][0], gps[j][0], sc, tc, sg, tg).astype(BF16)
        yb = _mm(w_ref[...], a).astype(BF16)
        outs[j][0] = yb
        s0 = s0 + _colsum_bf(yb)
        s1 = s1 + _colsum_bf(yb * yb)
    _accum_stats(sto_ref, _stats_rows([s0, s1], cout))


def _pre(cps, gps, st, bn, w, count_prev):
    nph = len(cps)
    B, C, T = cps[0].shape
    cout = w.shape[0]
    gc, bc, gg, bg = bn
    dspec = pl.BlockSpec((1, C, T), lambda b: (b, 0, 0))
    pspec = pl.BlockSpec((1, C), lambda b: (0, 0))
    ospec = pl.BlockSpec((1, cout, T), lambda b: (b, 0, 0))
    oshape = jax.ShapeDtypeStruct((B, cout, T), BF16)
    res = pl.pallas_call(
        functools.partial(_pre_body, nph, float(count_prev)),
        grid=(B,),
        in_specs=[dspec] * (2 * nph) + [
            pl.BlockSpec((8, C), lambda b: (0, 0)),
            pspec, pspec, pspec, pspec,
            pl.BlockSpec((cout, C), lambda b: (0, 0))],
        out_specs=[ospec] * nph + [pl.BlockSpec((8, cout), lambda b: (0, 0))],
        out_shape=[oshape] * nph + [jax.ShapeDtypeStruct((8, cout), F32)],
        compiler_params=_cp(),
    )(*cps, *gps, st, gc[None], bc[None], gg[None], bg[None], w)
    return res[:nph], res[nph]


def _logit_body(nph, count_prev, *refs):
    ys = refs[:nph]
    st_ref, pg_ref, pb_ref, w_ref, b_ref = refs[nph:nph + 5]
    o_ref = refs[nph + 5]
    inv = 1.0 / count_prev
    sc, tc = _bn_coeffs(st_ref, pg_ref, pb_ref, 0, inv)
    for j in range(nph):
        y = (ys[j][0] * sc[:, None] + tc[:, None]).astype(BF16)
        o_ref[0, j] = _mm(w_ref[...], y) + b_ref[0][:, None]


def _logit(ys, st, pg, pb, w, bias, count_prev):
    nph = len(ys)
    B, C, T = ys[0].shape
    cout = w.shape[0]
    pspec = pl.BlockSpec((1, C), lambda b: (0, 0))
    return pl.pallas_call(
        functools.partial(_logit_body, nph, float(count_prev)),
        grid=(B,),
        in_specs=[pl.BlockSpec((1, C, T), lambda b: (b, 0, 0))] * nph + [
            pl.BlockSpec((8, C), lambda b: (0, 0)),
            pspec, pspec,
            pl.BlockSpec((cout, C), lambda b: (0, 0)),
            pl.BlockSpec((1, cout), lambda b: (0, 0))],
        out_specs=pl.BlockSpec((1, nph, cout, T), lambda b: (b, 0, 0, 0)),
        out_shape=jax.ShapeDtypeStruct((B, nph, cout, T), F32),
        compiler_params=pltpu.CompilerParams(
            vmem_limit_bytes=100 * 1024 * 1024,
            dimension_semantics=("parallel",)),
    )(*ys, st, pg[None], pb[None], w, bias[None])


def _enc_w(w):
    return jnp.transpose(w, (2, 0, 1))


def _gen_w(w):
    return jnp.transpose(w, (2, 1, 0))[::-1]


def kernel(input, speaker, params):
    p = params
    B = input.shape[0]

    enc = p['encoder']
    C0 = input.shape[1]
    T = input.shape[2]
    x8 = jnp.transpose(input.reshape(B, C0, T // 8, 8), (0, 3, 1, 2))
    tout = (T - 4) // 2 + 1
    n_out = T // 8
    cps, gps, st = _enc1(x8, _enc_w(enc[0]['conv_w']),
                         _enc_w(enc[0]['gate_w']), tout, n_out)
    touts = [tout, (tout - 4) // 2 + 1, ((tout - 4) // 2 + 1 - 4) // 2 + 1]
    n_outs = [n_out, n_out, touts[2]]
    for li, (lp_prev, lp) in enumerate(zip(enc[:2], enc[1:])):
        count_prev = B * touts[li]
        bn = (lp_prev['conv_bn_g'], lp_prev['conv_bn_b'],
              lp_prev['gate_bn_g'], lp_prev['gate_bn_b'])
        cps, gps, st = _enc_fused(cps, gps, st, bn, _enc_w(lp['conv_w']),
                                  _enc_w(lp['gate_w']), touts[li + 1],
                                  n_outs[li + 1], count_prev)

    lp_prev = enc[2]
    bn = (lp_prev['conv_bn_g'], lp_prev['conv_bn_b'],
          lp_prev['gate_bn_g'], lp_prev['gate_bn_b'])
    T = touts[2]
    z, stl = _latent(cps[0], gps[0], st, bn, p['latent_w'][:, :, 0], B * T)
    idx = _vq_argmin(z, stl, p['latent_bn_g'], p['latent_bn_b'],
                     p['codebook'], B * T)

    nflat = B * T
    npad = -nflat % 256
    idxf = jnp.pad(idx.reshape(nflat), (0, npad)).astype(jnp.int32)
    d_lat = p['codebook'].shape[1]
    cb_p = jnp.pad(p['codebook'], ((0, 0), (0, 128 - d_lat)))
    emb_p = jnp.pad(p['speaker_emb'],
                    ((0, -p['speaker_emb'].shape[0] % 8),
                     (0, 128 - p['speaker_emb'].shape[1])))
    zq_rows, h = _sc_gather(cb_p, idxf, speaker.astype(jnp.int32), emb_p)
    h = h[:, :p['speaker_emb'].shape[1]]
    x = jnp.transpose(zq_rows[:nflat, :d_lat].reshape(B, T, d_lat), (0, 2, 1))

    gen = p['generator']
    lp = gen[0]
    ce, co, ge, go, st = _gen1(x, h, _gen_w(lp['conv_w']),
                               _gen_w(lp['gate_w']), lp['cond_w'])
    cps, gps = [ce, co], [ge, go]

    def _bn_of(lp_):
        return (lp_['conv_bn_g'], lp_['conv_bn_b'],
                lp_['gate_bn_g'], lp_['gate_bn_b'])

    cps, gps, st = _gen_fused(cps, gps, st, _bn_of(gen[0]),
                              _gen_w(gen[1]['conv_w']).astype(BF16),
                              _gen_w(gen[1]['gate_w']).astype(BF16),
                              h, gen[1]['cond_w'], B * 2 * T)

    cps, gps, st = _gen_fused(cps, gps, st, _bn_of(gen[1]),
                              _gen_w(gen[2]['conv_w']).astype(BF16),
                              _gen_w(gen[2]['gate_w']).astype(BF16),
                              h, gen[2]['cond_w'], B * 4 * T)
    ys, stp = _pre(cps, gps, st, _bn_of(gen[2]),
                   p['pre_w'][:, :, 0].astype(BF16), B * 8 * T)
    out4 = _logit(ys, stp, p['pre_bn_g'], p['pre_bn_b'],
                  p['logit_w'][:, :, 0].astype(BF16), p['logit_b'],
                  B * len(ys) * T)
    nph, cout = out4.shape[1], out4.shape[2]
    return lax.reshape(out4, (B, cout, nph * T), dimensions=(0, 2, 3, 1))


# speedup vs baseline: 1.0906x; 1.0906x over previous
import functools

import jax
import jax.numpy as jnp
from jax import lax
from jax.experimental import pallas as pl
from jax.experimental.pallas import tpu as pltpu
from jax.experimental.pallas import tpu_sc as plsc

F32 = jnp.float32
BF16 = jnp.bfloat16
EPS = 1e-5


def _cp():
    return pltpu.CompilerParams(vmem_limit_bytes=100 * 1024 * 1024)


def _mm(a, b, precision=None):
    return lax.dot_general(a, b, (((1,), (0,)), ((), ())),
                           preferred_element_type=F32, precision=precision)


def _stats_rows(pairs, cout):
    rows = [v[None, :] for v in pairs]
    rows.append(jnp.zeros((8 - len(rows), cout), F32))
    return jnp.concatenate(rows, axis=0)


def _colsum(q):
    return jnp.sum(q, axis=1)


def _accum_stats(st_ref, s):
    b = pl.program_id(0)

    @pl.when(b == 0)
    def _():
        st_ref[...] = s

    @pl.when(b != 0)
    def _():
        st_ref[...] = st_ref[...] + s


def _bn_coeffs(st_ref, g_ref, b_ref, row, inv):
    m = st_ref[row] * inv
    v = st_ref[row + 1] * inv - m * m
    s = g_ref[0] * lax.rsqrt(v + EPS)
    t = b_ref[0] - m * s
    return s, t


def _gate(c, g, sc, tc, sg, tg):
    cn = c * sc[:, None] + tc[:, None]
    gn = g * sg[:, None] + tg[:, None]
    return jnp.tanh(cn) * jax.nn.sigmoid(gn)


def _conv_stats(c, g):
    return _stats_rows([_colsum(c), _colsum(c * c), _colsum(g), _colsum(g * g)], c.shape[0])


def _phase_conv(p_list, w_ref, n_out, cond=None):
    P = len(p_list)
    Q = P // 2
    cin = p_list[0].shape[0]
    zcol = jnp.zeros((cin, 1), F32)
    shifted = {}

    def shift(j):
        if j not in shifted:
            shifted[j] = jnp.concatenate([p_list[j][:, 1:], zcol], axis=1)
        return shifted[j]

    qs = []
    for r in range(Q):
        acc = None
        for k in range(4):
            j = (2 * r + k) % P
            src = shift(j) if (2 * r + k) >= P else p_list[j]
            term = _mm(w_ref[k], src[:, :n_out])
            acc = term if acc is None else acc + term
        if cond is not None:
            acc = acc + cond
        qs.append(acc)
    return qs


def _zero_tail(q, valid):
    if valid >= q.shape[1]:
        return q
    ii = lax.broadcasted_iota(jnp.int32, q.shape, 1)
    return jnp.where(ii < valid, q, 0.0)


def _phase_valids(tout, Q, n_out):
    return [min(n_out, -(-(tout - r) // Q)) for r in range(Q)]


def _enc1_body(tout, x_ref, wc_ref, wg_ref, *out_refs):
    P = x_ref.shape[1]
    p_list = [x_ref[0, j] for j in range(P)]
    Q = P // 2
    n_out = out_refs[0].shape[2]
    valids = _phase_valids(tout, Q, n_out)
    qc = _phase_conv(p_list, wc_ref, n_out)
    qg = _phase_conv(p_list, wg_ref, n_out)
    qc = [_zero_tail(q, v) for q, v in zip(qc, valids)]
    qg = [_zero_tail(q, v) for q, v in zip(qg, valids)]
    for i, q in enumerate(qc + qg):
        out_refs[i][0] = q
    out_refs[-1][0] = _phase_stats(qc, qg)


def _phase_stats(qc, qg):
    cout = qc[0].shape[0]
    zs = jnp.zeros((cout,), F32)
    sums = [zs, zs, zs, zs]
    for q in qc:
        sums[0] = sums[0] + _colsum(q)
        sums[1] = sums[1] + _colsum(q * q)
    for q in qg:
        sums[2] = sums[2] + _colsum(q)
        sums[3] = sums[3] + _colsum(q * q)
    return _stats_rows(sums, cout)


def _enc1(x8, wc, wg, tout, n_out):
    B, P, C, n = x8.shape
    cout = wc.shape[1]
    Q = P // 2
    wspec = pl.BlockSpec((4, cout, C), lambda b: (0, 0, 0))
    ospec = pl.BlockSpec((1, cout, n_out), lambda b: (b, 0, 0))
    oshape = jax.ShapeDtypeStruct((B, cout, n_out), F32)
    res = pl.pallas_call(
        functools.partial(_enc1_body, tout),
        grid=(B,),
        in_specs=[pl.BlockSpec((1, P, C, n), lambda b: (b, 0, 0, 0)),
                  wspec, wspec],
        out_specs=[ospec] * (2 * Q) + [pl.BlockSpec((1, 8, cout),
                                                    lambda b: (b, 0, 0))],
        out_shape=[oshape] * (2 * Q) + [jax.ShapeDtypeStruct((B, 8, cout),
                                                             F32)],
        compiler_params=pltpu.CompilerParams(
            vmem_limit_bytes=100 * 1024 * 1024,
            dimension_semantics=("parallel",)),
    )(x8, wc, wg)
    return res[:Q], res[Q:2 * Q], res[2 * Q]


def _enc_body(tout, count_prev, nph, n_out, *refs):
    cps = refs[:nph]
    gps = refs[nph:2 * nph]
    st_ref, gc_ref, bc_ref, gg_ref, bg_ref, wc_ref, wg_ref = \
        refs[2 * nph:2 * nph + 7]
    out_refs = refs[2 * nph + 7:]
    inv = 1.0 / count_prev
    st = st_ref[...]
    if st.ndim == 3:
        st = jnp.sum(st, axis=0)
    sc, tc = _bn_coeffs(st, gc_ref, bc_ref, 0, inv)
    sg, tg = _bn_coeffs(st, gg_ref, bg_ref, 2, inv)
    p_list = [_gate(cps[j][0], gps[j][0], sc, tc, sg, tg) for j in range(nph)]
    Q = nph // 2
    valids = _phase_valids(tout, Q, n_out)
    qc = _phase_conv(p_list, wc_ref, n_out)
    qg = _phase_conv(p_list, wg_ref, n_out)
    qc = [_zero_tail(q, v) for q, v in zip(qc, valids)]
    qg = [_zero_tail(q, v) for q, v in zip(qg, valids)]
    for i, q in enumerate(qc + qg):
        out_refs[i][0] = q
    _accum_stats(out_refs[-1], _phase_stats(qc, qg))


def _enc_fused(cps, gps, st, bn, wc, wg, tout, n_out, count_prev):
    nph = len(cps)
    B, C, n = cps[0].shape
    cout = wc.shape[1]
    Q = nph // 2
    gc, bc, gg, bg = bn
    wspec = pl.BlockSpec((4, cout, C), lambda b: (0, 0, 0))
    dspec = pl.BlockSpec((1, C, n), lambda b: (b, 0, 0))
    pspec = pl.BlockSpec((1, C), lambda b: (0, 0))
    ospec = pl.BlockSpec((1, cout, n_out), lambda b: (b, 0, 0))
    oshape = jax.ShapeDtypeStruct((B, cout, n_out), F32)
    sspec = (pl.BlockSpec((B, 8, C), lambda b: (0, 0, 0)) if st.ndim == 3
             else pl.BlockSpec((8, C), lambda b: (0, 0)))
    res = pl.pallas_call(
        functools.partial(_enc_body, tout, float(count_prev), nph, n_out),
        grid=(B,),
        in_specs=[dspec] * (2 * nph) + [
            sspec, pspec, pspec, pspec, pspec, wspec, wspec],
        out_specs=[ospec] * (2 * Q) + [pl.BlockSpec((8, cout),
                                                    lambda b: (0, 0))],
        out_shape=[oshape] * (2 * Q) + [jax.ShapeDtypeStruct((8, cout), F32)],
        compiler_params=_cp(),
    )(*cps, *gps, st, gc[None], bc[None], gg[None], bg[None], wc, wg)
    return res[:Q], res[Q:2 * Q], res[2 * Q]


def _latent_body(count_prev, cp_ref, gp_ref, st_ref, gc_ref, bc_ref,
                 gg_ref, bg_ref, w_ref, z_ref, sto_ref):
    inv = 1.0 / count_prev
    sc, tc = _bn_coeffs(st_ref, gc_ref, bc_ref, 0, inv)
    sg, tg = _bn_coeffs(st_ref, gg_ref, bg_ref, 2, inv)
    act = _gate(cp_ref[0], gp_ref[0], sc, tc, sg, tg)
    z = _mm(w_ref[...], act)
    z_ref[0] = z
    _accum_stats(sto_ref, _stats_rows(
        [_colsum(z), _colsum(z * z)], z.shape[0]))


def _latent(cp, gp, st, bn, w, count_prev):
    B, C, T = cp.shape
    cout = w.shape[0]
    gc, bc, gg, bg = bn
    dspec = pl.BlockSpec((1, C, T), lambda b: (b, 0, 0))
    pspec = pl.BlockSpec((1, C), lambda b: (0, 0))
    return pl.pallas_call(
        functools.partial(_latent_body, float(count_prev)),
        grid=(B,),
        in_specs=[dspec, dspec, pl.BlockSpec((8, C), lambda b: (0, 0)),
                  pspec, pspec, pspec, pspec,
                  pl.BlockSpec((cout, C), lambda b: (0, 0))],
        out_specs=[pl.BlockSpec((1, cout, T), lambda b: (b, 0, 0)),
                   pl.BlockSpec((8, cout), lambda b: (0, 0))],
        out_shape=[jax.ShapeDtypeStruct((B, cout, T), F32),
                   jax.ShapeDtypeStruct((8, cout), F32)],
        compiler_params=_cp(),
    )(cp, gp, st, gc[None], bc[None], gg[None], bg[None], w)


def _vq_body(count, z_ref, st_ref, lg_ref, lb_ref, cb_ref, idx_ref):
    inv = 1.0 / count
    m = st_ref[0] * inv
    v = st_ref[1] * inv - m * m
    sc = lg_ref[0] * lax.rsqrt(v + EPS)
    tc = lb_ref[0] - m * sc
    z = z_ref[0] * sc[:, None] + tc[:, None]
    cb = cb_ref[...]
    s = _mm(cb, z, precision=lax.Precision.HIGHEST)
    cn2 = jnp.sum(cb * cb, axis=1)
    val = cn2[:, None] - 2.0 * s
    mn = jnp.min(val, axis=0)
    ii = lax.broadcasted_iota(jnp.int32, val.shape, 0)
    idx = jnp.min(jnp.where(val <= mn[None, :], ii, jnp.int32(1 << 30)), axis=0)
    idx_ref[0, 0] = idx


def _vq_argmin(z, st, lg, lb, cb, count):
    B, D, T = z.shape
    n = cb.shape[0]
    return pl.pallas_call(
        functools.partial(_vq_body, float(count)),
        grid=(B,),
        in_specs=[pl.BlockSpec((1, D, T), lambda b: (b, 0, 0)),
                  pl.BlockSpec((8, D), lambda b: (0, 0)),
                  pl.BlockSpec((1, D), lambda b: (0, 0)),
                  pl.BlockSpec((1, D), lambda b: (0, 0)),
                  pl.BlockSpec((n, D), lambda b: (0, 0))],
        out_specs=pl.BlockSpec((1, 1, T), lambda b: (b, 0, 0)),
        out_shape=jax.ShapeDtypeStruct((B, 1, T), jnp.int32),
        compiler_params=_cp(),
    )(z, st, lg[None], lb[None], cb)


def _sc_gather(cb, idx, spk, emb):
    info = plsc.get_sparse_core_info()
    nc, ns = info.num_cores, info.num_subcores
    nw = nc * ns
    nidx = idx.shape[0]
    bpw = nidx // nw
    d = cb.shape[1]
    bsp = spk.shape[0]
    mesh = plsc.VectorSubcoreMesh(core_axis_name="c", subcore_axis_name="s")

    @functools.partial(
        pl.kernel,
        out_type=[jax.ShapeDtypeStruct((nidx, d), F32),
                  jax.ShapeDtypeStruct((bsp, emb.shape[1]), F32)],
        mesh=mesh,
        scratch_types=[pltpu.VMEM((bpw,), jnp.int32),
                       pltpu.VMEM((bpw, d), F32),
                       pltpu.VMEM((bsp,), jnp.int32),
                       pltpu.VMEM((bsp, emb.shape[1]), F32),
                       pltpu.SemaphoreType.DMA],
    )
    def k(cb_hbm, idx_hbm, spk_hbm, emb_hbm, zq_hbm, h_hbm,
          idx_v, rows_v, sidx_v, srows_v, sem):
        wid = lax.axis_index("s") * nc + lax.axis_index("c")
        base = wid * bpw
        pltpu.sync_copy(idx_hbm.at[pl.ds(base, bpw)], idx_v)
        pltpu.async_copy(cb_hbm.at[idx_v], rows_v, sem).wait()
        pltpu.sync_copy(rows_v, zq_hbm.at[pl.ds(base, bpw)])

        @pl.when(wid == 0)
        def _():
            pltpu.sync_copy(spk_hbm, sidx_v)
            pltpu.async_copy(emb_hbm.at[sidx_v], srows_v, sem).wait()
            pltpu.sync_copy(srows_v, h_hbm)

    return k(cb, idx, spk, emb)


def _gen1_body(mc_ref, mg_ref, x_ref, h_ref, cw_ref,
               ce_ref, co_ref, ge_ref, go_ref, st_ref):
    x = x_ref[0]
    cin, t = x.shape
    z1 = jnp.zeros((cin, 1), F32)
    xm = jnp.concatenate([z1, x[:, :t - 1]], axis=1)
    xp = jnp.concatenate([x[:, 1:], z1], axis=1)
    cond = _mm(cw_ref[...], h_ref[0, 0][:, None])
    ce = _mm(mc_ref[0], xm) + _mm(mc_ref[2], x) + cond
    co = _mm(mc_ref[1], x) + _mm(mc_ref[3], xp) + cond
    ge = _mm(mg_ref[0], xm) + _mm(mg_ref[2], x) + cond
    go = _mm(mg_ref[1], x) + _mm(mg_ref[3], xp) + cond
    ce_ref[0] = ce.astype(BF16)
    co_ref[0] = co.astype(BF16)
    ge_ref[0] = ge.astype(BF16)
    go_ref[0] = go.astype(BF16)
    s = _stats_rows(
        [_colsum(ce) + _colsum(co),
         _colsum(ce * ce) + _colsum(co * co),
         _colsum(ge) + _colsum(go),
         _colsum(ge * ge) + _colsum(go * go)], ce.shape[0])
    _accum_stats(st_ref, s)


def _gen1(x, h, mc, mg, cw):
    B, cin, T = x.shape
    cout = cw.shape[0]
    h3 = h[:, None, :]
    wspec = pl.BlockSpec((4, cout, cin), lambda b: (0, 0, 0))
    ospec = pl.BlockSpec((1, cout, T), lambda b: (b, 0, 0))
    oshape = jax.ShapeDtypeStruct((B, cout, T), BF16)
    return pl.pallas_call(
        _gen1_body,
        grid=(B,),
        in_specs=[wspec, wspec,
                  pl.BlockSpec((1, cin, T), lambda b: (b, 0, 0)),
                  pl.BlockSpec((1, 1, h.shape[1]), lambda b: (b, 0, 0)),
                  pl.BlockSpec((cout, h.shape[1]), lambda b: (0, 0))],
        out_specs=[ospec, ospec, ospec, ospec,
                   pl.BlockSpec((8, cout), lambda b: (0, 0))],
        out_shape=[oshape, oshape, oshape, oshape,
                   jax.ShapeDtypeStruct((8, cout), F32)],
        compiler_params=_cp(),
    )(mc, mg, x, h3, cw)


def _genf_body(nph, count_prev, *refs):
    cps = refs[:nph]
    gps = refs[nph:2 * nph]
    st_ref, gc_ref, bc_ref, gg_ref, bg_ref, mc_ref, mg_ref, h_ref, cw_ref = \
        refs[2 * nph:2 * nph + 9]
    outs = refs[2 * nph + 9:2 * nph + 9 + 4 * nph]
    sto_ref = refs[2 * nph + 9 + 4 * nph]

    inv = 1.0 / count_prev
    sc, tc = _bn_coeffs(st_ref, gc_ref, bc_ref, 0, inv)
    sg, tg = _bn_coeffs(st_ref, gg_ref, bg_ref, 2, inv)
    p = [_gate(cps[j][0], gps[j][0], sc, tc, sg, tg).astype(BF16)
         for j in range(nph)]
    cond = _mm(cw_ref[...], h_ref[0, 0][:, None])
    qc = _tconv_phases(p, mc_ref, cond)
    qg = _tconv_phases(p, mg_ref, cond)
    for i, q in enumerate(qc + qg):
        outs[i][0] = q.astype(BF16)
    _accum_stats(sto_ref, _phase_stats(qc, qg))


def _act_body(nph, count_prev, *refs):
    cps = refs[:nph]
    gps = refs[nph:2 * nph]
    st_ref, gc_ref, bc_ref, gg_ref, bg_ref = refs[2 * nph:2 * nph + 5]
    outs = refs[2 * nph + 5:]
    inv = 1.0 / count_prev
    sc, tc = _bn_coeffs(st_ref, gc_ref, bc_ref, 0, inv)
    sg, tg = _bn_coeffs(st_ref, gg_ref, bg_ref, 2, inv)
    for j in range(nph):
        outs[j][0] = _gate(cps[j][0], gps[j][0], sc, tc, sg, tg).astype(BF16)


def _act_phases(cps, gps, st, bn, count_prev):
    nph = len(cps)
    B, C, T = cps[0].shape
    gc, bc, gg, bg = bn
    dspec = pl.BlockSpec((1, C, T), lambda b: (b, 0, 0))
    pspec = pl.BlockSpec((1, C), lambda b: (0, 0))
    oshape = jax.ShapeDtypeStruct((B, C, T), BF16)
    return pl.pallas_call(
        functools.partial(_act_body, nph, float(count_prev)),
        grid=(B,),
        in_specs=[dspec] * (2 * nph) + [
            pl.BlockSpec((8, C), lambda b: (0, 0)),
            pspec, pspec, pspec, pspec],
        out_specs=[dspec] * nph,
        out_shape=[oshape] * nph,
        compiler_params=_cp(),
    )(*cps, *gps, st, gc[None], bc[None], gg[None], bg[None])


def _tconv_phases(p_list, m_ref, cond):
    nph = len(p_list)
    cin = p_list[0].shape[0]
    z1 = jnp.zeros((cin, 1), p_list[0].dtype)
    pm = jnp.concatenate([z1, p_list[nph - 1][:, :-1]], axis=1)
    pp = jnp.concatenate([p_list[0][:, 1:], z1], axis=1)
    qs = []
    for j in range(nph):
        a = pm if j == 0 else p_list[j - 1]
        b = pp if j == nph - 1 else p_list[j + 1]
        qs.append(_mm(m_ref[0], a) + _mm(m_ref[2], p_list[j]) + cond)
        qs.append(_mm(m_ref[1], p_list[j]) + _mm(m_ref[3], b) + cond)
    return qs


def _gen_half_body(nph, *refs):
    ps = refs[:nph]
    m_ref, h_ref, cw_ref = refs[nph:nph + 3]
    outs = refs[nph + 3:nph + 3 + 2 * nph]
    sto_ref = refs[nph + 3 + 2 * nph]
    cond = _mm(cw_ref[...], h_ref[0, 0][:, None])
    qs = _tconv_phases([ps[j][0] for j in range(nph)], m_ref, cond)
    for i, q in enumerate(qs):
        outs[i][0] = q
    cout = qs[0].shape[0]
    zs = jnp.zeros((cout,), F32)
    s0, s1 = zs, zs
    for q in qs:
        s0 = s0 + _colsum(q)
        s1 = s1 + _colsum(q * q)
    _accum_stats(sto_ref, _stats_rows([s0, s1], cout))


def _gen_half(ps, m, h, cw):
    nph = len(ps)
    B, C, T = ps[0].shape
    cout = cw.shape[0]
    h3 = h[:, None, :]
    dspec = pl.BlockSpec((1, C, T), lambda b: (b, 0, 0))
    ospec = pl.BlockSpec((1, cout, T), lambda b: (b, 0, 0))
    oshape = jax.ShapeDtypeStruct((B, cout, T), F32)
    res = pl.pallas_call(
        functools.partial(_gen_half_body, nph),
        grid=(B,),
        in_specs=[dspec] * nph + [
            pl.BlockSpec((4, cout, C), lambda b: (0, 0, 0)),
            pl.BlockSpec((1, 1, h.shape[1]), lambda b: (b, 0, 0)),
            pl.BlockSpec((cout, h.shape[1]), lambda b: (0, 0))],
        out_specs=[ospec] * (2 * nph) + [pl.BlockSpec((8, cout),
                                                      lambda b: (0, 0))],
        out_shape=[oshape] * (2 * nph) + [jax.ShapeDtypeStruct((8, cout), F32)],
        compiler_params=_cp(),
    )(*ps, m, h3, cw)
    return res[:2 * nph], res[2 * nph]


def _gen_fused(cps, gps, st, bn, mc, mg, h, cw, count_prev):
    nph = len(cps)
    B, C, T = cps[0].shape
    cout = cw.shape[0]
    gc, bc, gg, bg = bn
    h3 = h[:, None, :]
    dspec = pl.BlockSpec((1, C, T), lambda b: (b, 0, 0))
    pspec = pl.BlockSpec((1, C), lambda b: (0, 0))
    wspec = pl.BlockSpec((4, cout, C), lambda b: (0, 0, 0))
    ospec = pl.BlockSpec((1, cout, T), lambda b: (b, 0, 0))
    oshape = jax.ShapeDtypeStruct((B, cout, T), BF16)
    res = pl.pallas_call(
        functools.partial(_genf_body, nph, float(count_prev)),
        grid=(B,),
        in_specs=[dspec] * (2 * nph) + [
            pl.BlockSpec((8, C), lambda b: (0, 0)),
            pspec, pspec, pspec, pspec, wspec, wspec,
            pl.BlockSpec((1, 1, h.shape[1]), lambda b: (b, 0, 0)),
            pl.BlockSpec((cout, h.shape[1]), lambda b: (0, 0))],
        out_specs=[ospec] * (4 * nph) + [pl.BlockSpec((8, cout),
                                                      lambda b: (0, 0))],
        out_shape=[oshape] * (4 * nph) + [jax.ShapeDtypeStruct((8, cout), F32)],
        compiler_params=_cp(),
    )(*cps, *gps, st, gc[None], bc[None], gg[None], bg[None], mc, mg, h3, cw)
    qs, sto = res[:4 * nph], res[4 * nph]
    return qs[:2 * nph], qs[2 * nph:], sto


def _pre_body(nph, count_prev, *refs):
    cps = refs[:nph]
    gps = refs[nph:2 * nph]
    st_ref, gc_ref, bc_ref, gg_ref, bg_ref, w_ref = \
        refs[2 * nph:2 * nph + 6]
    outs = refs[2 * nph + 6:2 * nph + 6 + nph]
    sto_ref = refs[2 * nph + 6 + nph]
    inv = 1.0 / count_prev
    sc, tc = _bn_coeffs(st_ref, gc_ref, bc_ref, 0, inv)
    sg, tg = _bn_coeffs(st_ref, gg_ref, bg_ref, 2, inv)
    cout = w_ref.shape[0]
    zs = jnp.zeros((cout,), F32)
    s0, s1 = zs, zs
    for j in range(nph):
        a = _gate(cps[j][0], gps[j][0], sc, tc, sg, tg).astype(BF16)
        y = _mm(w_ref[...], a)
        outs[j][0] = y.astype(BF16)
        s0 = s0 + _colsum(y)
        s1 = s1 + _colsum(y * y)
    _accum_stats(sto_ref, _stats_rows([s0, s1], cout))


def _pre(cps, gps, st, bn, w, count_prev):
    nph = len(cps)
    B, C, T = cps[0].shape
    cout = w.shape[0]
    gc, bc, gg, bg = bn
    dspec = pl.BlockSpec((1, C, T), lambda b: (b, 0, 0))
    pspec = pl.BlockSpec((1, C), lambda b: (0, 0))
    ospec = pl.BlockSpec((1, cout, T), lambda b: (b, 0, 0))
    oshape = jax.ShapeDtypeStruct((B, cout, T), BF16)
    res = pl.pallas_call(
        functools.partial(_pre_body, nph, float(count_prev)),
        grid=(B,),
        in_specs=[dspec] * (2 * nph) + [
            pl.BlockSpec((8, C), lambda b: (0, 0)),
            pspec, pspec, pspec, pspec,
            pl.BlockSpec((cout, C), lambda b: (0, 0))],
        out_specs=[ospec] * nph + [pl.BlockSpec((8, cout), lambda b: (0, 0))],
        out_shape=[oshape] * nph + [jax.ShapeDtypeStruct((8, cout), F32)],
        compiler_params=_cp(),
    )(*cps, *gps, st, gc[None], bc[None], gg[None], bg[None], w)
    return res[:nph], res[nph]


def _logit_body(nph, count_prev, *refs):
    ys = refs[:nph]
    st_ref, pg_ref, pb_ref, w_ref, b_ref = refs[nph:nph + 5]
    o_ref = refs[nph + 5]
    inv = 1.0 / count_prev
    sc, tc = _bn_coeffs(st_ref, pg_ref, pb_ref, 0, inv)
    for j in range(nph):
        y = (ys[j][0] * sc[:, None] + tc[:, None]).astype(BF16)
        o_ref[0, j] = _mm(w_ref[...], y) + b_ref[0][:, None]


def _logit(ys, st, pg, pb, w, bias, count_prev):
    nph = len(ys)
    B, C, T = ys[0].shape
    cout = w.shape[0]
    pspec = pl.BlockSpec((1, C), lambda b: (0, 0))
    return pl.pallas_call(
        functools.partial(_logit_body, nph, float(count_prev)),
        grid=(B,),
        in_specs=[pl.BlockSpec((1, C, T), lambda b: (b, 0, 0))] * nph + [
            pl.BlockSpec((8, C), lambda b: (0, 0)),
            pspec, pspec,
            pl.BlockSpec((cout, C), lambda b: (0, 0)),
            pl.BlockSpec((1, cout), lambda b: (0, 0))],
        out_specs=pl.BlockSpec((1, nph, cout, T), lambda b: (b, 0, 0, 0)),
        out_shape=jax.ShapeDtypeStruct((B, nph, cout, T), F32),
        compiler_params=pltpu.CompilerParams(
            vmem_limit_bytes=100 * 1024 * 1024,
            dimension_semantics=("parallel",)),
    )(*ys, st, pg[None], pb[None], w, bias[None])


def _enc_w(w):
    return jnp.transpose(w, (2, 0, 1))


def _gen_w(w):
    return jnp.transpose(w, (2, 1, 0))[::-1]


def kernel(input, speaker, params):
    p = params
    B = input.shape[0]

    enc = p['encoder']
    C0 = input.shape[1]
    T = input.shape[2]
    x8 = jnp.transpose(input.reshape(B, C0, T // 8, 8), (0, 3, 1, 2))
    tout = (T - 4) // 2 + 1
    n_out = T // 8
    cps, gps, st = _enc1(x8, _enc_w(enc[0]['conv_w']),
                         _enc_w(enc[0]['gate_w']), tout, n_out)
    touts = [tout, (tout - 4) // 2 + 1, ((tout - 4) // 2 + 1 - 4) // 2 + 1]
    n_outs = [n_out, n_out, touts[2]]
    for li, (lp_prev, lp) in enumerate(zip(enc[:2], enc[1:])):
        count_prev = B * touts[li]
        bn = (lp_prev['conv_bn_g'], lp_prev['conv_bn_b'],
              lp_prev['gate_bn_g'], lp_prev['gate_bn_b'])
        cps, gps, st = _enc_fused(cps, gps, st, bn, _enc_w(lp['conv_w']),
                                  _enc_w(lp['gate_w']), touts[li + 1],
                                  n_outs[li + 1], count_prev)

    lp_prev = enc[2]
    bn = (lp_prev['conv_bn_g'], lp_prev['conv_bn_b'],
          lp_prev['gate_bn_g'], lp_prev['gate_bn_b'])
    T = touts[2]
    z, stl = _latent(cps[0], gps[0], st, bn, p['latent_w'][:, :, 0], B * T)
    idx = _vq_argmin(z, stl, p['latent_bn_g'], p['latent_bn_b'],
                     p['codebook'], B * T)

    nflat = B * T
    npad = -nflat % 256
    idxf = jnp.pad(idx.reshape(nflat), (0, npad)).astype(jnp.int32)
    d_lat = p['codebook'].shape[1]
    cb_p = jnp.pad(p['codebook'], ((0, 0), (0, 128 - d_lat)))
    emb_p = jnp.pad(p['speaker_emb'],
                    ((0, -p['speaker_emb'].shape[0] % 8),
                     (0, 128 - p['speaker_emb'].shape[1])))
    zq_rows, h = _sc_gather(cb_p, idxf, speaker.astype(jnp.int32), emb_p)
    h = h[:, :p['speaker_emb'].shape[1]]
    x = jnp.transpose(zq_rows[:nflat, :d_lat].reshape(B, T, d_lat), (0, 2, 1))

    gen = p['generator']
    lp = gen[0]
    ce, co, ge, go, st = _gen1(x, h, _gen_w(lp['conv_w']),
                               _gen_w(lp['gate_w']), lp['cond_w'])
    cps, gps = [ce, co], [ge, go]

    def _bn_of(lp_):
        return (lp_['conv_bn_g'], lp_['conv_bn_b'],
                lp_['gate_bn_g'], lp_['gate_bn_b'])

    cps, gps, st = _gen_fused(cps, gps, st, _bn_of(gen[0]),
                              _gen_w(gen[1]['conv_w']).astype(BF16),
                              _gen_w(gen[1]['gate_w']).astype(BF16),
                              h, gen[1]['cond_w'], B * 2 * T)

    cps, gps, st = _gen_fused(cps, gps, st, _bn_of(gen[1]),
                              _gen_w(gen[2]['conv_w']).astype(BF16),
                              _gen_w(gen[2]['gate_w']).astype(BF16),
                              h, gen[2]['cond_w'], B * 4 * T)
    ys, stp = _pre(cps, gps, st, _bn_of(gen[2]),
                   p['pre_w'][:, :, 0].astype(BF16), B * 8 * T)
    out4 = _logit(ys, stp, p['pre_bn_g'], p['pre_bn_b'],
                  p['logit_w'][:, :, 0].astype(BF16), p['logit_b'],
                  B * len(ys) * T)
    nph, cout = out4.shape[1], out4.shape[2]
    return lax.reshape(out4, (B, cout, nph * T), dimensions=(0, 2, 3, 1))
